# baseline (speedup 1.0000x reference)
import functools
import jax
import jax.numpy as jnp
from jax import lax
from jax.experimental import pallas as pl
from jax.experimental.pallas import tpu as pltpu
from jax.experimental.pallas import tpu_sc as plsc

N = 10000
E = 320000
F = 16
IN_C = 128

NC = 2
NS = 16
NT = NC * NS
L = 16

EPT = E // NT
FT = 4
EG = NT // (F // FT)
EPG = E // EG
CE = 8000
NK = EPG // CE

_MESH = plsc.VectorSubcoreMesh(core_axis_name="c", subcore_axis_name="s",
                               num_cores=NC, num_subcores=NS)


_SC_PARAMS = pltpu.CompilerParams(needs_layout_passes=False)


@functools.partial(
    pl.kernel,
    out_type=jax.ShapeDtypeStruct((NT, N), jnp.float32),
    mesh=_MESH,
    compiler_params=_SC_PARAMS,
    scratch_types=[
        pltpu.VMEM((EPT,), jnp.int32),
        pltpu.VMEM((EPT,), jnp.float32),
        pltpu.VMEM((N,), jnp.float32),
    ],
)
def _deg_kernel(col_hbm, ea_hbm, out_hbm, col_v, ea_v, deg_v):
    c = lax.axis_index("c")
    s = lax.axis_index("s")
    wid = s * NC + c
    base = wid * EPT
    pltpu.sync_copy(col_hbm.at[pl.ds(base, EPT)], col_v)
    pltpu.sync_copy(ea_hbm.at[pl.ds(base, EPT)], ea_v)

    zer = jnp.zeros((L,), jnp.float32)

    @plsc.parallel_loop(0, N // L, unroll=8)
    def _zero(i):
        deg_v[pl.ds(i * L, L)] = zer

    @plsc.parallel_loop(0, EPT // L, unroll=8)
    def _scat(i):
        sl = pl.ds(i * L, L)
        plsc.addupdate_scatter(deg_v, [col_v[sl]], ea_v[sl])

    pltpu.sync_copy(deg_v, out_hbm.at[wid])


@functools.partial(
    pl.kernel,
    out_type=jax.ShapeDtypeStruct((EG, F, N), jnp.float32),
    mesh=_MESH,
    compiler_params=_SC_PARAMS,
    scratch_types=[
        [pltpu.VMEM((N,), jnp.float32) for _ in range(FT)],
        [pltpu.VMEM((N,), jnp.float32) for _ in range(FT)],
        [pltpu.VMEM((CE,), jnp.int32) for _ in range(2)],
        [pltpu.VMEM((CE,), jnp.int32) for _ in range(2)],
        [pltpu.VMEM((CE,), jnp.float32) for _ in range(2)],
        [pltpu.SemaphoreType.DMA for _ in range(2)],
    ],
)
def _agg_kernel(h_hbm, row_hbm, col_hbm, ea_hbm, out_hbm,
                ts, accs, row_v, col_v, ea_v, sems):
    c = lax.axis_index("c")
    s = lax.axis_index("s")
    fg = s % FT
    eg = (s // FT) * NC + c
    f0 = fg * FT

    for j in range(FT):
        pltpu.sync_copy(h_hbm.at[f0 + j], ts[j])

    zer = jnp.zeros((L,), jnp.float32)

    @plsc.parallel_loop(0, N // L, unroll=8)
    def _zeroacc(i):
        sl = pl.ds(i * L, L)
        for j in range(FT):
            accs[j][sl] = zer

    ebase = eg * EPG

    def _start(k, b):
        off = ebase + k * CE
        return [
            pltpu.async_copy(row_hbm.at[pl.ds(off, CE)], row_v[b], sems[b]),
            pltpu.async_copy(col_hbm.at[pl.ds(off, CE)], col_v[b], sems[b]),
            pltpu.async_copy(ea_hbm.at[pl.ds(off, CE)], ea_v[b], sems[b]),
        ]

    copies = _start(0, 0)
    for k in range(NK):
        b = k % 2
        nxt = _start(k + 1, 1 - b) if k + 1 < NK else None
        for cp in copies:
            cp.wait()

        @plsc.parallel_loop(0, CE // L)
        def _edges(i):
            sl = pl.ds(i * L, L)
            rows = row_v[b][sl]
            cols = col_v[b][sl]
            eav = ea_v[b][sl]
            for j in range(FT):
                vals = plsc.load_gather(ts[j], [rows]) * eav
                plsc.addupdate_scatter(accs[j], [cols], vals)

        copies = nxt

    for j in range(FT):
        pltpu.sync_copy(accs[j], out_hbm.at[eg, f0 + j])


def _mm1_prep_body(x_ref, w_ref, degp_ref, h1_ref, h1s_ref, dinv_ref,
                   selfw_ref):
    h1 = lax.dot_general(
        w_ref[...], x_ref[...], (((0,), (1,)), ((), ())),
        preferred_element_type=jnp.float32)
    deg = 1.0 + jnp.sum(degp_ref[...], axis=0, keepdims=True)
    dinv = lax.rsqrt(deg)
    h1_ref[...] = h1
    h1s_ref[...] = h1 * dinv
    dinv_ref[...] = dinv
    selfw_ref[...] = 1.0 / deg


def _layer_body(agg_ref, h_ref, dinv_ref, selfw_ref, b_ref, w_ref,
                h2_ref, h2s_ref):
    aggsum = jnp.sum(agg_ref[...], axis=0)
    dinv = dinv_ref[...]
    hf = jnp.maximum(
        aggsum * dinv + h_ref[...] * selfw_ref[...] + b_ref[...],
        0.0)
    h2 = lax.dot_general(
        w_ref[...], hf, (((0,), (0,)), ((), ())),
        preferred_element_type=jnp.float32)
    h2_ref[...] = h2
    h2s_ref[...] = h2 * dinv


def _head_body(agg_ref, h_ref, dinv_ref, selfw_ref, b_ref, wo_ref, bo_ref,
               o_ref):
    aggsum = jnp.sum(agg_ref[...], axis=0)
    hf = jnp.maximum(
        aggsum * dinv_ref[...] + h_ref[...] * selfw_ref[...] + b_ref[...],
        0.0)
    o_ref[...] = lax.dot_general(
        hf, wo_ref[...], (((0,), (0,)), ((), ())),
        preferred_element_type=jnp.float32) + bo_ref[...]


def kernel(x, edge_index, edge_attr, y, batch, W1, b1, W2, b2, Wo, bo):
    row = edge_index[0]
    col = edge_index[1]

    degp = _deg_kernel(col, edge_attr)

    h1T, h1sT, dinv, selfw = pl.pallas_call(
        _mm1_prep_body,
        out_shape=[jax.ShapeDtypeStruct((F, N), jnp.float32),
                   jax.ShapeDtypeStruct((F, N), jnp.float32),
                   jax.ShapeDtypeStruct((1, N), jnp.float32),
                   jax.ShapeDtypeStruct((1, N), jnp.float32)],
    )(x, W1, degp)

    agg1 = _agg_kernel(h1sT, row, col, edge_attr)

    h2T, h2sT = pl.pallas_call(
        _layer_body,
        out_shape=[jax.ShapeDtypeStruct((F, N), jnp.float32),
                   jax.ShapeDtypeStruct((F, N), jnp.float32)],
    )(agg1, h1T, dinv, selfw, b1.reshape(F, 1), W2)

    agg2 = _agg_kernel(h2sT, row, col, edge_attr)

    out = pl.pallas_call(
        _head_body,
        out_shape=jax.ShapeDtypeStruct((N, 1), jnp.float32),
    )(agg2, h2T, dinv, selfw, b2.reshape(F, 1), Wo, bo.reshape(1, 1))

    return out


# speedup vs baseline: 1.1175x; 1.1175x over previous
import functools
import jax
import jax.numpy as jnp
from jax import lax
from jax.experimental import pallas as pl
from jax.experimental.pallas import tpu as pltpu
from jax.experimental.pallas import tpu_sc as plsc

N = 10000
E = 320000
F = 16
IN_C = 128

NC = 2
NS = 16
NT = NC * NS
L = 16

EPT = E // NT
FT = 4
EG = NT // (F // FT)
EPG = E // EG
CE = 8000
NK = EPG // CE

_MESH = plsc.VectorSubcoreMesh(core_axis_name="c", subcore_axis_name="s",
                               num_cores=NC, num_subcores=NS)


_SC_PARAMS = pltpu.CompilerParams(needs_layout_passes=False)


@functools.partial(
    pl.kernel,
    out_type=jax.ShapeDtypeStruct((NT, N), jnp.float32),
    mesh=_MESH,
    compiler_params=_SC_PARAMS,
    scratch_types=[
        pltpu.VMEM((EPT,), jnp.int32),
        pltpu.VMEM((EPT,), jnp.float32),
        pltpu.VMEM((N,), jnp.float32),
        pltpu.SemaphoreType.DMA,
    ],
)
def _deg_kernel(col_hbm, ea_hbm, out_hbm, col_v, ea_v, deg_v, sem):
    c = lax.axis_index("c")
    s = lax.axis_index("s")
    wid = s * NC + c
    base = wid * EPT
    cp1 = pltpu.async_copy(col_hbm.at[pl.ds(base, EPT)], col_v, sem)
    cp2 = pltpu.async_copy(ea_hbm.at[pl.ds(base, EPT)], ea_v, sem)

    zer = jnp.zeros((L,), jnp.float32)

    @plsc.parallel_loop(0, N // L, unroll=8)
    def _zero(i):
        deg_v[pl.ds(i * L, L)] = zer

    cp1.wait()
    cp2.wait()

    @plsc.parallel_loop(0, EPT // L, unroll=2)
    def _scat(i):
        sl = pl.ds(i * L, L)
        plsc.addupdate_scatter(deg_v, [col_v[sl]], ea_v[sl])

    pltpu.sync_copy(deg_v, out_hbm.at[wid])


@functools.partial(
    pl.kernel,
    out_type=jax.ShapeDtypeStruct((EG, F, N), jnp.float32),
    mesh=_MESH,
    compiler_params=_SC_PARAMS,
    scratch_types=[
        [pltpu.VMEM((N,), jnp.float32) for _ in range(FT)],
        [pltpu.VMEM((N,), jnp.float32) for _ in range(FT)],
        [pltpu.VMEM((CE,), jnp.int32) for _ in range(2)],
        [pltpu.VMEM((CE,), jnp.int32) for _ in range(2)],
        [pltpu.VMEM((CE,), jnp.float32) for _ in range(2)],
        [pltpu.SemaphoreType.DMA for _ in range(2)],
        pltpu.SemaphoreType.DMA,
    ],
)
def _agg_kernel(h_hbm, row_hbm, col_hbm, ea_hbm, out_hbm,
                ts, accs, row_v, col_v, ea_v, sems, tsem):
    c = lax.axis_index("c")
    s = lax.axis_index("s")
    fg = s % FT
    eg = (s // FT) * NC + c
    f0 = fg * FT

    ebase = eg * EPG

    def _start(k, b):
        off = ebase + k * CE
        return [
            pltpu.async_copy(row_hbm.at[pl.ds(off, CE)], row_v[b], sems[b]),
            pltpu.async_copy(col_hbm.at[pl.ds(off, CE)], col_v[b], sems[b]),
            pltpu.async_copy(ea_hbm.at[pl.ds(off, CE)], ea_v[b], sems[b]),
        ]

    copies = _start(0, 0)
    tcopies = [pltpu.async_copy(h_hbm.at[f0 + j], ts[j], tsem)
               for j in range(FT)]

    zer = jnp.zeros((L,), jnp.float32)

    @plsc.parallel_loop(0, N // L, unroll=8)
    def _zeroacc(i):
        sl = pl.ds(i * L, L)
        for j in range(FT):
            accs[j][sl] = zer

    for cp in tcopies:
        cp.wait()
    for k in range(NK):
        b = k % 2
        nxt = _start(k + 1, 1 - b) if k + 1 < NK else None
        for cp in copies:
            cp.wait()

        @plsc.parallel_loop(0, CE // L, unroll=2)
        def _edges(i):
            sl = pl.ds(i * L, L)
            rows = row_v[b][sl]
            cols = col_v[b][sl]
            eav = ea_v[b][sl]
            for j in range(FT):
                vals = plsc.load_gather(ts[j], [rows]) * eav
                plsc.addupdate_scatter(accs[j], [cols], vals)

        copies = nxt

    for j in range(FT):
        pltpu.sync_copy(accs[j], out_hbm.at[eg, f0 + j])


def _mm1_prep_body(x_ref, w_ref, degp_ref, h1_ref, h1s_ref, dinv_ref,
                   selfw_ref):
    h1 = lax.dot_general(
        w_ref[...], x_ref[...], (((0,), (1,)), ((), ())),
        preferred_element_type=jnp.float32)
    deg = 1.0 + jnp.sum(degp_ref[...], axis=0, keepdims=True)
    dinv = lax.rsqrt(deg)
    h1_ref[...] = h1
    h1s_ref[...] = h1 * dinv
    dinv_ref[...] = dinv
    selfw_ref[...] = 1.0 / deg


def _layer_body(agg_ref, h_ref, dinv_ref, selfw_ref, b_ref, w_ref,
                h2_ref, h2s_ref):
    aggsum = jnp.sum(agg_ref[...], axis=0)
    dinv = dinv_ref[...]
    hf = jnp.maximum(
        aggsum * dinv + h_ref[...] * selfw_ref[...] + b_ref[...],
        0.0)
    h2 = lax.dot_general(
        w_ref[...], hf, (((0,), (0,)), ((), ())),
        preferred_element_type=jnp.float32)
    h2_ref[...] = h2
    h2s_ref[...] = h2 * dinv


def _head_body(agg_ref, h_ref, dinv_ref, selfw_ref, b_ref, wo_ref, bo_ref,
               o_ref):
    aggsum = jnp.sum(agg_ref[...], axis=0)
    hf = jnp.maximum(
        aggsum * dinv_ref[...] + h_ref[...] * selfw_ref[...] + b_ref[...],
        0.0)
    o_ref[...] = lax.dot_general(
        hf, wo_ref[...], (((0,), (0,)), ((), ())),
        preferred_element_type=jnp.float32) + bo_ref[...]


def kernel(x, edge_index, edge_attr, y, batch, W1, b1, W2, b2, Wo, bo):
    row = edge_index[0]
    col = edge_index[1]

    degp = _deg_kernel(col, edge_attr)

    h1T, h1sT, dinv, selfw = pl.pallas_call(
        _mm1_prep_body,
        out_shape=[jax.ShapeDtypeStruct((F, N), jnp.float32),
                   jax.ShapeDtypeStruct((F, N), jnp.float32),
                   jax.ShapeDtypeStruct((1, N), jnp.float32),
                   jax.ShapeDtypeStruct((1, N), jnp.float32)],
    )(x, W1, degp)

    agg1 = _agg_kernel(h1sT, row, col, edge_attr)

    h2T, h2sT = pl.pallas_call(
        _layer_body,
        out_shape=[jax.ShapeDtypeStruct((F, N), jnp.float32),
                   jax.ShapeDtypeStruct((F, N), jnp.float32)],
    )(agg1, h1T, dinv, selfw, b1.reshape(F, 1), W2)

    agg2 = _agg_kernel(h2sT, row, col, edge_attr)

    out = pl.pallas_call(
        _head_body,
        out_shape=jax.ShapeDtypeStruct((N, 1), jnp.float32),
    )(agg2, h2T, dinv, selfw, b2.reshape(F, 1), Wo, bo.reshape(1, 1))

    return out
